# Initial kernel scaffold; baseline (speedup 1.0000x reference)
import jax
import jax.numpy as jnp
from jax.experimental import pallas as pl


def kernel(x, edge_index, W1, b1, W2, b2):
    raise NotImplementedError("write your pallas kernel here")


# speedup vs baseline: 7.8401x; 7.8401x over previous
import functools

import jax
import jax.numpy as jnp
from jax import lax
from jax.experimental import pallas as pl
from jax.experimental.pallas import tpu as pltpu
from jax.experimental.pallas import tpu_sc as plsc

N = 10000
E = 320000
D = 128

NC = 2
NS = 16
NW = NC * NS
CHUNK = 128
E_PAD = NW * 80 * CHUNK
NCHUNKS = E_PAD // CHUNK
CH_PER_W = NCHUNKS // NW
RPT = 632
N_PAD = NS * RPT


_mesh = plsc.VectorSubcoreMesh(core_axis_name="c", subcore_axis_name="s")


def _tile_rows(s):
    base = s * RPT
    return base, min(RPT, N_PAD - base)


def _zero_fill(ref, nrows, ncols, value=0.0):
    @pl.loop(0, nrows)
    def _(i):
        @pl.loop(0, ncols, step=16)
        def _(j):
            ref[i, pl.ds(j, 16)] = jnp.full((16,), value, jnp.float32)


@functools.partial(
    pl.kernel,
    out_type=jax.ShapeDtypeStruct((NC, N_PAD, 16), jnp.float32),
    mesh=_mesh,
    scratch_types=[
        pltpu.VMEM((CHUNK,), jnp.int32),
        pltpu.VMEM((CHUNK, 16), jnp.float32),
        pltpu.VMEM((CHUNK, 16), jnp.float32),
        pltpu.VMEM_SHARED((N_PAD, 16), jnp.float32),
    ],
)
def _deg_kernel(dst_hbm, out_hbm, idx_v, ones_v, zb_v, acc_sh):
    c = lax.axis_index("c")
    s = lax.axis_index("s")
    wid = c * NS + s
    base = wid * CH_PER_W * CHUNK

    @pl.loop(0, CHUNK)
    def _(i):
        ones_v.at[pl.ds(i, 1), pl.ds(0, 16)][...] = jnp.ones((1, 16), jnp.float32)
        zb_v.at[pl.ds(i, 1), pl.ds(0, 16)][...] = jnp.zeros((1, 16), jnp.float32)

    for k in range(0, RPT, CHUNK):
        n = min(CHUNK, RPT - k)
        pltpu.sync_copy(zb_v.at[pl.ds(0, n)], acc_sh.at[pl.ds(s * RPT + k, n)])
    plsc.subcore_barrier()

    @pl.loop(0, CH_PER_W)
    def _(j):
        pltpu.sync_copy(dst_hbm.at[pl.ds(base + j * CHUNK, CHUNK)], idx_v)
        pltpu.sync_copy(ones_v, acc_sh.at[idx_v], add=True)

    plsc.subcore_barrier()
    for k in range(0, RPT, CHUNK):
        n = min(CHUNK, RPT - k)
        pltpu.sync_copy(acc_sh.at[pl.ds(s * RPT + k, n)], zb_v.at[pl.ds(0, n)])
        pltpu.sync_copy(zb_v.at[pl.ds(0, n)],
                        out_hbm.at[c, pl.ds(s * RPT + k, n)])


@functools.partial(
    pl.kernel,
    out_type=jax.ShapeDtypeStruct((NC, N_PAD, D), jnp.float32),
    mesh=_mesh,
    scratch_types=[
        pltpu.VMEM((CHUNK,), jnp.int32),
        pltpu.VMEM((CHUNK,), jnp.int32),
        pltpu.VMEM((CHUNK, D), jnp.float32),
        pltpu.VMEM_SHARED((N_PAD, D), jnp.float32),
        pltpu.SemaphoreType.DMA,
    ],
)
def _agg_kernel(z_hbm, src_hbm, dst_hbm, out_hbm, src_v, dst_v, rows_v, acc_sh, sem):
    c = lax.axis_index("c")
    s = lax.axis_index("s")
    wid = c * NS + s
    base = wid * CH_PER_W * CHUNK

    @pl.loop(0, CHUNK)
    def _(i):
        @pl.loop(0, D, step=16)
        def _(jj):
            rows_v.at[pl.ds(i, 1), pl.ds(jj, 16)][...] = (
                jnp.zeros((1, 16), jnp.float32))

    for k in range(0, RPT, CHUNK):
        n = min(CHUNK, RPT - k)
        pltpu.sync_copy(rows_v.at[pl.ds(0, n)],
                        acc_sh.at[pl.ds(s * RPT + k, n)])
    plsc.subcore_barrier()

    @pl.loop(0, CH_PER_W)
    def _(j):
        pltpu.sync_copy(src_hbm.at[pl.ds(base + j * CHUNK, CHUNK)], src_v)
        pltpu.sync_copy(dst_hbm.at[pl.ds(base + j * CHUNK, CHUNK)], dst_v)
        pltpu.async_copy(z_hbm.at[src_v], rows_v, sem).wait()
        pltpu.sync_copy(rows_v, acc_sh.at[dst_v], add=True)

    plsc.subcore_barrier()
    for k in range(0, RPT, CHUNK):
        n = min(CHUNK, RPT - k)
        pltpu.sync_copy(acc_sh.at[pl.ds(s * RPT + k, n)],
                        rows_v.at[pl.ds(0, n)])
        pltpu.sync_copy(rows_v.at[pl.ds(0, n)],
                        out_hbm.at[c, pl.ds(s * RPT + k, n)])


def _mm_body(x_ref, w_ref, o_ref):
    o_ref[...] = lax.dot_general(
        x_ref[...], w_ref[...], (((1,), (0,)), ((), ())),
        precision=lax.Precision.HIGHEST, preferred_element_type=jnp.float32)


_mm = pl.pallas_call(
    _mm_body, out_shape=jax.ShapeDtypeStruct((N, D), jnp.float32))


def _scale_body(degp_ref, y_ref, z_ref, dinv_ref):
    deg = degp_ref[0, :, 0:1] + degp_ref[1, :, 0:1] + 1.0
    dinv = lax.rsqrt(deg)
    dinv_ref[...] = dinv
    z_ref[:N, :] = y_ref[...] * dinv[:N]
    z_ref[N:, :] = jnp.zeros((N_PAD - N, D), jnp.float32)


_scale = pl.pallas_call(
    _scale_body,
    out_shape=(jax.ShapeDtypeStruct((N_PAD, D), jnp.float32),
               jax.ShapeDtypeStruct((N_PAD, 1), jnp.float32)))


def _mid_body(acc_ref, z_ref, dinv_ref, b1_ref, w2_ref, z2_ref):
    dinv = dinv_ref[...]
    h = (acc_ref[0] + acc_ref[1] + z_ref[...]) * dinv + b1_ref[...]
    h = jnp.maximum(h, 0.0)
    y2 = lax.dot_general(h, w2_ref[...], (((1,), (0,)), ((), ())),
                         precision=lax.Precision.HIGHEST,
                         preferred_element_type=jnp.float32)
    z2_ref[:N, :] = (y2 * dinv)[:N]
    z2_ref[N:, :] = jnp.zeros((N_PAD - N, D), jnp.float32)


_mid = pl.pallas_call(
    _mid_body, out_shape=jax.ShapeDtypeStruct((N_PAD, D), jnp.float32))


def _final_body(acc_ref, z_ref, dinv_ref, b2_ref, o_ref):
    acc = acc_ref[...]
    o_ref[...] = ((acc[0, :N] + acc[1, :N] + z_ref[...][:N])
                  * dinv_ref[...][:N] + b2_ref[...])


_final = pl.pallas_call(
    _final_body, out_shape=jax.ShapeDtypeStruct((N, D), jnp.float32))


def kernel(x, edge_index, W1, b1, W2, b2):
    pad = jnp.full((E_PAD - E,), N, jnp.int32)
    src1 = jnp.concatenate([edge_index[0], pad])
    dst1 = jnp.concatenate([edge_index[1], pad])
    degp = _deg_kernel(dst1)
    y1 = _mm(x, W1)
    z1, dinv = _scale(degp, y1)
    acc1 = _agg_kernel(z1, src1, dst1)
    z2 = _mid(acc1, z1, dinv, b1.reshape(1, D), W2)
    acc2 = _agg_kernel(z2, src1, dst1)
    return _final(acc2, z2, dinv, b2.reshape(1, D))


# speedup vs baseline: 9.7471x; 1.2432x over previous
import functools

import jax
import jax.numpy as jnp
from jax import lax
from jax.experimental import pallas as pl
from jax.experimental.pallas import tpu as pltpu
from jax.experimental.pallas import tpu_sc as plsc

N = 10000
E = 320000
D = 128

NC = 2
NS = 16
NW = NC * NS
CHUNK = 128
E_PAD = NW * 80 * CHUNK
NCHUNKS = E_PAD // CHUNK
CH_PER_W = NCHUNKS // NW
RPT = 632
N_PAD = NS * RPT


_mesh = plsc.VectorSubcoreMesh(core_axis_name="c", subcore_axis_name="s")


def _tile_rows(s):
    base = s * RPT
    return base, min(RPT, N_PAD - base)


def _zero_fill(ref, nrows, ncols, value=0.0):
    @pl.loop(0, nrows)
    def _(i):
        @pl.loop(0, ncols, step=16)
        def _(j):
            ref[i, pl.ds(j, 16)] = jnp.full((16,), value, jnp.float32)


@functools.partial(
    pl.kernel,
    out_type=jax.ShapeDtypeStruct((NC, N_PAD, 16), jnp.float32),
    mesh=_mesh,
    scratch_types=[
        pltpu.VMEM((CHUNK,), jnp.int32),
        pltpu.VMEM((CHUNK, 16), jnp.float32),
        pltpu.VMEM((CHUNK, 16), jnp.float32),
        pltpu.VMEM_SHARED((N_PAD, 16), jnp.float32),
    ],
)
def _deg_kernel(dst_hbm, out_hbm, idx_v, ones_v, zb_v, acc_sh):
    c = lax.axis_index("c")
    s = lax.axis_index("s")
    wid = c * NS + s
    base = wid * CH_PER_W * CHUNK

    @pl.loop(0, CHUNK)
    def _(i):
        ones_v.at[pl.ds(i, 1), pl.ds(0, 16)][...] = jnp.ones((1, 16), jnp.float32)
        zb_v.at[pl.ds(i, 1), pl.ds(0, 16)][...] = jnp.zeros((1, 16), jnp.float32)

    for k in range(0, RPT, CHUNK):
        n = min(CHUNK, RPT - k)
        pltpu.sync_copy(zb_v.at[pl.ds(0, n)], acc_sh.at[pl.ds(s * RPT + k, n)])
    plsc.subcore_barrier()

    @pl.loop(0, CH_PER_W)
    def _(j):
        pltpu.sync_copy(dst_hbm.at[pl.ds(base + j * CHUNK, CHUNK)], idx_v)
        pltpu.sync_copy(ones_v, acc_sh.at[idx_v], add=True)

    plsc.subcore_barrier()
    for k in range(0, RPT, CHUNK):
        n = min(CHUNK, RPT - k)
        pltpu.sync_copy(acc_sh.at[pl.ds(s * RPT + k, n)], zb_v.at[pl.ds(0, n)])
        pltpu.sync_copy(zb_v.at[pl.ds(0, n)],
                        out_hbm.at[c, pl.ds(s * RPT + k, n)])


@functools.partial(
    pl.kernel,
    out_type=jax.ShapeDtypeStruct((NC, N_PAD, D), jnp.float32),
    mesh=_mesh,
    scratch_types=[
        pltpu.VMEM((CHUNK,), jnp.int32),
        pltpu.VMEM((CHUNK,), jnp.int32),
        pltpu.VMEM((CHUNK,), jnp.int32),
        pltpu.VMEM((CHUNK,), jnp.int32),
        pltpu.VMEM((CHUNK, D), jnp.float32),
        pltpu.VMEM((CHUNK, D), jnp.float32),
        pltpu.VMEM_SHARED((N_PAD, D), jnp.float32),
        pltpu.SemaphoreType.DMA,
        pltpu.SemaphoreType.DMA,
        pltpu.SemaphoreType.DMA,
        pltpu.SemaphoreType.DMA,
    ],
)
def _agg_kernel(z_hbm, src_hbm, dst_hbm, out_hbm,
                src0, src1, dst0, dst1, rows0, rows1, acc_sh,
                g0, g1, i0, i1, *, _b=None):
    c = lax.axis_index("c")
    s = lax.axis_index("s")
    wid = c * NS + s
    base = wid * CH_PER_W * CHUNK
    srcb, dstb, rowsb = (src0, src1), (dst0, dst1), (rows0, rows1)
    gsem, isem = (g0, g1), (i0, i1)

    def idx_start(j, b):
        off = base + j * CHUNK
        pltpu.make_async_copy(src_hbm.at[pl.ds(off, CHUNK)], srcb[b], isem[b]).start()
        pltpu.make_async_copy(dst_hbm.at[pl.ds(off, CHUNK)], dstb[b], isem[b]).start()

    def idx_wait(j, b):
        off = base + j * CHUNK
        pltpu.make_async_copy(src_hbm.at[pl.ds(off, CHUNK)], srcb[b], isem[b]).wait()
        pltpu.make_async_copy(dst_hbm.at[pl.ds(off, CHUNK)], dstb[b], isem[b]).wait()

    def gather_start(b):
        pltpu.make_async_copy(z_hbm.at[srcb[b]], rowsb[b], gsem[b]).start()

    def gather_wait(b):
        pltpu.make_async_copy(z_hbm.at[srcb[b]], rowsb[b], gsem[b]).wait()

    def scatter(b):
        pltpu.sync_copy(rowsb[b], acc_sh.at[dstb[b]], add=True)

    idx_start(0, 0)
    idx_start(1, 1)
    @pl.loop(0, CHUNK)
    def _(i):
        @pl.loop(0, D, step=16)
        def _(jj):
            rows0.at[pl.ds(i, 1), pl.ds(jj, 16)][...] = (
                jnp.zeros((1, 16), jnp.float32))

    for k in range(0, RPT, CHUNK):
        n = min(CHUNK, RPT - k)
        pltpu.sync_copy(rows0.at[pl.ds(0, n)],
                        acc_sh.at[pl.ds(s * RPT + k, n)])
    idx_wait(0, 0)
    gather_start(0)
    idx_wait(1, 1)
    gather_start(1)
    plsc.subcore_barrier()

    @pl.loop(0, (CH_PER_W - 2) // 2)
    def _(t):
        j = t * 2
        for b in range(2):
            gather_wait(b)
            idx_start(j + 2 + b, b)
            scatter(b)
            idx_wait(j + 2 + b, b)
            gather_start(b)

    for b in range(2):
        gather_wait(b)
        scatter(b)

    plsc.subcore_barrier()
    for k in range(0, RPT, CHUNK):
        n = min(CHUNK, RPT - k)
        pltpu.sync_copy(acc_sh.at[pl.ds(s * RPT + k, n)],
                        rows0.at[pl.ds(0, n)])
        pltpu.sync_copy(rows0.at[pl.ds(0, n)],
                        out_hbm.at[c, pl.ds(s * RPT + k, n)])


def _mm_body(x_ref, w_ref, o_ref):
    o_ref[...] = lax.dot_general(
        x_ref[...], w_ref[...], (((1,), (0,)), ((), ())),
        precision=lax.Precision.HIGHEST, preferred_element_type=jnp.float32)


_mm = pl.pallas_call(
    _mm_body, out_shape=jax.ShapeDtypeStruct((N, D), jnp.float32))


def _scale_body(degp_ref, y_ref, z_ref, dinv_ref):
    deg = degp_ref[0, :, 0:1] + degp_ref[1, :, 0:1] + 1.0
    dinv = lax.rsqrt(deg)
    dinv_ref[...] = dinv
    z_ref[:N, :] = y_ref[...] * dinv[:N]
    z_ref[N:, :] = jnp.zeros((N_PAD - N, D), jnp.float32)


_scale = pl.pallas_call(
    _scale_body,
    out_shape=(jax.ShapeDtypeStruct((N_PAD, D), jnp.float32),
               jax.ShapeDtypeStruct((N_PAD, 1), jnp.float32)))


def _mid_body(acc_ref, z_ref, dinv_ref, b1_ref, w2_ref, z2_ref):
    dinv = dinv_ref[...]
    h = (acc_ref[0] + acc_ref[1] + z_ref[...]) * dinv + b1_ref[...]
    h = jnp.maximum(h, 0.0)
    y2 = lax.dot_general(h, w2_ref[...], (((1,), (0,)), ((), ())),
                         precision=lax.Precision.HIGHEST,
                         preferred_element_type=jnp.float32)
    z2_ref[:N, :] = (y2 * dinv)[:N]
    z2_ref[N:, :] = jnp.zeros((N_PAD - N, D), jnp.float32)


_mid = pl.pallas_call(
    _mid_body, out_shape=jax.ShapeDtypeStruct((N_PAD, D), jnp.float32))


def _final_body(acc_ref, z_ref, dinv_ref, b2_ref, o_ref):
    acc = acc_ref[...]
    o_ref[...] = ((acc[0, :N] + acc[1, :N] + z_ref[...][:N])
                  * dinv_ref[...][:N] + b2_ref[...])


_final = pl.pallas_call(
    _final_body, out_shape=jax.ShapeDtypeStruct((N, D), jnp.float32))


def kernel(x, edge_index, W1, b1, W2, b2):
    pad = jnp.full((E_PAD - E,), N, jnp.int32)
    src1 = jnp.concatenate([edge_index[0], pad])
    dst1 = jnp.concatenate([edge_index[1], pad])
    degp = _deg_kernel(dst1)
    y1 = _mm(x, W1)
    z1, dinv = _scale(degp, y1)
    acc1 = _agg_kernel(z1, src1, dst1)
    z2 = _mid(acc1, z1, dinv, b1.reshape(1, D), W2)
    acc2 = _agg_kernel(z2, src1, dst1)
    return _final(acc2, z2, dinv, b2.reshape(1, D))
